# Initial kernel scaffold; baseline (speedup 1.0000x reference)
import jax
import jax.numpy as jnp
from jax.experimental import pallas as pl


def kernel(agent_ids, W_agent, W_temp):
    raise NotImplementedError("write your pallas kernel here")


# speedup vs baseline: 1.2574x; 1.2574x over previous
import functools

import jax
import jax.numpy as jnp
from jax import lax
from jax.experimental import pallas as pl
from jax.experimental.pallas import tpu as pltpu
from jax.experimental.pallas import tpu_sc as plsc

BATCH = 16384
SEQ = 50
D_AGENT = 64
D_TEMP = 16
D_OUT = D_AGENT + D_TEMP
BS = BATCH * SEQ

NUM_WORKERS = 32
ROWS_PER_W = BS // NUM_WORKERS
CHUNK = 128
CHUNKS_PER_W = ROWS_PER_W // CHUNK
B_PER_W = BATCH // NUM_WORKERS
EB = 32
EB_STEPS = B_PER_W // EB


def _body(ids2d, col0, wa, wt, out, idx_v, rows_v, idxb_v, trows_v, exp_v, sem):
    wid = lax.axis_index("s") * 2 + lax.axis_index("c")
    row0 = wid * ROWS_PER_W

    pltpu.sync_copy(ids2d.at[pl.ds(wid * CHUNKS_PER_W, CHUNKS_PER_W)], idx_v)

    def agent_step(j, carry):
        pltpu.async_copy(wa.at[idx_v.at[j]], rows_v, sem).wait()
        pltpu.sync_copy(
            rows_v,
            out.at[pl.ds(row0 + j * CHUNK, CHUNK), pl.ds(0, D_AGENT)],
        )
        return carry

    lax.fori_loop(0, CHUNKS_PER_W, agent_step, 0)

    b0 = wid * B_PER_W
    pltpu.sync_copy(col0.at[pl.ds(wid * (B_PER_W // 128), B_PER_W // 128)], idxb_v)
    for t in range(B_PER_W // 128):
        pltpu.async_copy(
            wt.at[idxb_v.at[t]], trows_v.at[pl.ds(t * 128, 128)], sem
        ).wait()

    def temp_chunk(c, carry):
        def expand_b(b, carry2):
            v = trows_v[c * EB + b, :]

            def put(s2, carry3):
                exp_v[b * SEQ + s2, :] = v
                return carry3

            lax.fori_loop(0, SEQ, put, 0)
            return carry2

        lax.fori_loop(0, EB, expand_b, 0)
        pltpu.sync_copy(
            exp_v,
            out.at[pl.ds(row0 + c * EB * SEQ, EB * SEQ), pl.ds(D_AGENT, D_TEMP)],
        )
        return carry

    lax.fori_loop(0, EB_STEPS, temp_chunk, 0)


@functools.partial(jax.jit, static_argnums=())
def kernel(agent_ids, W_agent, W_temp):
    ids2d = agent_ids.reshape(BS // CHUNK, CHUNK).astype(jnp.int32)
    col0 = agent_ids[:, 0].reshape(BATCH // 128, 128).astype(jnp.int32)

    run = pl.kernel(
        _body,
        out_type=jax.ShapeDtypeStruct((BS, D_OUT), jnp.float32),
        mesh=plsc.VectorSubcoreMesh(core_axis_name="c", subcore_axis_name="s"),
        scratch_types=[
            pltpu.VMEM((CHUNKS_PER_W, CHUNK), jnp.int32),
            pltpu.VMEM((CHUNK, D_AGENT), jnp.float32),
            pltpu.VMEM((B_PER_W // 128, 128), jnp.int32),
            pltpu.VMEM((B_PER_W, D_TEMP), jnp.float32),
            pltpu.VMEM((EB * SEQ, D_TEMP), jnp.float32),
            pltpu.SemaphoreType.DMA,
        ],
        compiler_params=pltpu.CompilerParams(use_tc_tiling_on_sc=False),
    )
    out = run(ids2d, col0, W_agent, W_temp)
    return out.reshape(BATCH, SEQ, D_OUT)


# speedup vs baseline: 1.4052x; 1.1175x over previous
import functools

import jax
import jax.numpy as jnp
from jax import lax
from jax.experimental import pallas as pl
from jax.experimental.pallas import tpu as pltpu
from jax.experimental.pallas import tpu_sc as plsc

BATCH = 16384
SEQ = 50
D_AGENT = 64
D_TEMP = 16
D_OUT = D_AGENT + D_TEMP
BS = BATCH * SEQ

NUM_WORKERS = 32
ROWS_PER_W = BS // NUM_WORKERS
CHUNK = 128
CHUNKS_PER_W = ROWS_PER_W // CHUNK
B_PER_W = BATCH // NUM_WORKERS
NBUF = 4


def _body(ids2d, col0, brel, wa, wt, out,
          idx_v, tidx_v, col0_v, brel_v, arows_v, trows_v, semA, semT):
    wid = lax.axis_index("s") * 2 + lax.axis_index("c")
    row0 = wid * ROWS_PER_W

    pltpu.sync_copy(ids2d.at[pl.ds(wid * CHUNKS_PER_W, CHUNKS_PER_W)], idx_v)
    pltpu.sync_copy(col0.at[pl.ds(wid * B_PER_W, B_PER_W)], col0_v)
    pltpu.sync_copy(brel, brel_v)

    def build_tidx(j, carry):
        for i in range(CHUNK // 16):
            b_rel = brel_v[j, pl.ds(i * 16, 16)]
            vals = plsc.load_gather(col0_v, [b_rel])
            tidx_v[j, pl.ds(i * 16, 16)] = vals
        return carry

    lax.fori_loop(0, CHUNKS_PER_W, build_tidx, 0)

    def copyA(slot, j):
        return pltpu.make_async_copy(wa.at[idx_v.at[j]], arows_v.at[slot],
                                     semA.at[slot])

    def copyT(slot, j):
        return pltpu.make_async_copy(wt.at[tidx_v.at[j]], trows_v.at[slot],
                                     semT.at[slot])

    for b in range(NBUF):
        copyA(b, b).start()
        copyT(b, b).start()

    def group(g, carry):
        for b in range(NBUF):
            j = g * NBUF + b
            copyA(b, j).wait()
            copyT(b, j).wait()
            pltpu.sync_copy(
                arows_v.at[b],
                out.at[pl.ds(row0 + j * CHUNK, CHUNK), pl.ds(0, D_AGENT)],
            )
            pltpu.sync_copy(
                trows_v.at[b],
                out.at[pl.ds(row0 + j * CHUNK, CHUNK), pl.ds(D_AGENT, D_TEMP)],
            )
            jn = jnp.minimum(j + NBUF, CHUNKS_PER_W - 1)
            copyA(b, jn).start()
            copyT(b, jn).start()
        return carry

    lax.fori_loop(0, CHUNKS_PER_W // NBUF, group, 0)

    for b in range(NBUF):
        copyA(b, CHUNKS_PER_W - 1).wait()
        copyT(b, CHUNKS_PER_W - 1).wait()


@functools.partial(jax.jit, static_argnums=())
def kernel(agent_ids, W_agent, W_temp):
    ids2d = agent_ids.reshape(BS // CHUNK, CHUNK).astype(jnp.int32)
    col0 = agent_ids[:, 0].astype(jnp.int32)
    brel = (jnp.arange(ROWS_PER_W, dtype=jnp.int32) // SEQ).reshape(
        CHUNKS_PER_W, CHUNK)

    run = pl.kernel(
        _body,
        out_type=jax.ShapeDtypeStruct((BS, D_OUT), jnp.float32),
        mesh=plsc.VectorSubcoreMesh(core_axis_name="c", subcore_axis_name="s"),
        scratch_types=[
            pltpu.VMEM((CHUNKS_PER_W, CHUNK), jnp.int32),
            pltpu.VMEM((CHUNKS_PER_W, CHUNK), jnp.int32),
            pltpu.VMEM((B_PER_W,), jnp.int32),
            pltpu.VMEM((CHUNKS_PER_W, CHUNK), jnp.int32),
            pltpu.VMEM((NBUF, CHUNK, D_AGENT), jnp.float32),
            pltpu.VMEM((NBUF, CHUNK, D_TEMP), jnp.float32),
            pltpu.SemaphoreType.DMA((NBUF,)),
            pltpu.SemaphoreType.DMA((NBUF,)),
        ],
        compiler_params=pltpu.CompilerParams(
            use_tc_tiling_on_sc=False, needs_layout_passes=False),
    )
    out = run(ids2d, col0, brel, W_agent, W_temp)
    return out.reshape(BATCH, SEQ, D_OUT)
